# baseline (speedup 1.0000x reference)
import functools

import jax
import jax.numpy as jnp
from jax import lax
from jax.experimental import pallas as pl
from jax.experimental.pallas import tpu as pltpu
from jax.experimental.pallas import tpu_sc as plsc

_ALPHA = 4
_NW = 32
_QROWS = 96


def _make_sc_pack(C, T, H, W, dtype):
    n = T // _ALPHA
    nf = C * T
    fpw = nf // _NW
    qpf = H // _QROWS
    cpw = fpw * qpf
    mesh = plsc.VectorSubcoreMesh(core_axis_name="c", subcore_axis_name="s")

    @functools.partial(
        pl.kernel,
        mesh=mesh,
        out_type=(
            jax.ShapeDtypeStruct((C * n * H, W), dtype),
            jax.ShapeDtypeStruct((C * T * H, W), dtype),
        ),
        scratch_types=[
            pltpu.VMEM((_QROWS, W), dtype),
            pltpu.VMEM((_QROWS, W), dtype),
            pltpu.SemaphoreType.DMA,
            pltpu.SemaphoreType.DMA,
            pltpu.SemaphoreType.DMA,
            pltpu.SemaphoreType.DMA,
        ],
    )
    def k(table_hbm, slow_hbm, fast_hbm, buf0, buf1, gs0, gs1, ss0, ss1):
        wid = lax.axis_index("s") * 2 + lax.axis_index("c")
        bufs = (buf0, buf1)
        gsems = (gs0, gs1)
        ssems = (ss0, ss1)
        f0 = wid * fpw

        def frame_info(j):
            f = f0 + j // qpf
            q = j % qpf
            t = f % T
            p = (5 * t + 20) // 21
            sel = (21 * p) // 5 == t
            row = f * H + q * _QROWS
            srow = ((f // T) * n + p) * H + q * _QROWS
            return row, sel, srow

        def gather(j, slot):
            row, _, _ = frame_info(j)
            return pltpu.make_async_copy(
                table_hbm.at[pl.ds(row, _QROWS)], bufs[slot], gsems[slot]
            )

        def fast_scatter(j, slot):
            row, _, _ = frame_info(j)
            return pltpu.make_async_copy(
                bufs[slot], fast_hbm.at[pl.ds(row, _QROWS)], ssems[slot]
            )

        def slow_scatter(j, slot):
            _, _, srow = frame_info(j)
            return pltpu.make_async_copy(
                bufs[slot], slow_hbm.at[pl.ds(srow, _QROWS)], ssems[slot]
            )

        def drain_scatters(j, slot):
            fast_scatter(j, slot).wait()
            _, sel, _ = frame_info(j)

            @pl.when(sel)
            def _():
                slow_scatter(j, slot).wait()

        gather(0, 0).start()
        for j in range(cpw):
            slot = j % 2
            gather(j, slot).wait()
            fast_scatter(j, slot).start()
            _, sel, _ = frame_info(j)

            @pl.when(sel)
            def _():
                slow_scatter(j, slot).start()

            if j + 1 < cpw:
                nslot = (j + 1) % 2
                if j >= 1:
                    drain_scatters(j - 1, nslot)
                gather(j + 1, nslot).start()
        drain_scatters(cpw - 2, (cpw - 2) % 2)
        drain_scatters(cpw - 1, (cpw - 1) % 2)

    return k


def kernel(frames):
    C, T, H, W = frames.shape
    n = T // _ALPHA
    table = frames.reshape(C * T * H, W)
    slow2d, fast2d = _make_sc_pack(C, T, H, W, frames.dtype)(table)
    return (slow2d.reshape(C, n, H, W), fast2d.reshape(C, T, H, W))


# speedup vs baseline: 1.1049x; 1.1049x over previous
import functools

import jax
import jax.numpy as jnp
from jax import lax
from jax.experimental import pallas as pl
from jax.experimental.pallas import tpu as pltpu
from jax.experimental.pallas import tpu_sc as plsc

_ALPHA = 4
_NW = 32
_QROWS = 96


def _make_sc_gather(C, T, H, W, dtype):
    n = T // _ALPHA
    n_sel = C * n
    qpf = H // _QROWS
    nq = n_sel * qpf
    qpw = nq // _NW
    mesh = plsc.VectorSubcoreMesh(core_axis_name="c", subcore_axis_name="s")

    @functools.partial(
        pl.kernel,
        mesh=mesh,
        out_type=jax.ShapeDtypeStruct((n_sel * H, W), dtype),
        scratch_types=[
            pltpu.VMEM((_QROWS, W), dtype),
            pltpu.VMEM((_QROWS, W), dtype),
            pltpu.SemaphoreType.DMA,
            pltpu.SemaphoreType.DMA,
            pltpu.SemaphoreType.DMA,
            pltpu.SemaphoreType.DMA,
        ],
    )
    def k(table_hbm, out_hbm, buf0, buf1, gs0, gs1, ss0, ss1):
        wid = lax.axis_index("s") * 2 + lax.axis_index("c")
        bufs = (buf0, buf1)
        gsems = (gs0, gs1)
        ssems = (ss0, ss1)

        def src_off(q):
            sel = q // qpf
            quarter = q % qpf
            frame = (sel // n) * T + (21 * (sel % n)) // 5
            return frame * H + quarter * _QROWS

        def gather(q, slot):
            return pltpu.make_async_copy(
                table_hbm.at[pl.ds(src_off(q), _QROWS)], bufs[slot], gsems[slot]
            )

        def scatter(q, slot):
            return pltpu.make_async_copy(
                bufs[slot], out_hbm.at[pl.ds(q * _QROWS, _QROWS)], ssems[slot]
            )

        q0 = wid * qpw
        gather(q0, 0).start()
        for b in range(qpw):
            slot = b % 2
            q = q0 + b
            gather(q, slot).wait()
            scatter(q, slot).start()
            if b + 1 < qpw:
                nslot = (b + 1) % 2
                if b >= 1:
                    scatter(q - 1, nslot).wait()
                gather(q + 1, nslot).start()
        scatter(q0 + qpw - 2, (qpw - 2) % 2).wait()
        scatter(q0 + qpw - 1, (qpw - 1) % 2).wait()

    return k


def _copy_body(in_ref, out_ref):
    out_ref[...] = in_ref[...]


def _tc_copy(frames):
    C, T, H, W = frames.shape
    tb = 16
    return pl.pallas_call(
        _copy_body,
        grid=(C, T // tb),
        in_specs=[pl.BlockSpec((1, tb, H, W), lambda c, t: (c, t, 0, 0))],
        out_specs=pl.BlockSpec((1, tb, H, W), lambda c, t: (c, t, 0, 0)),
        out_shape=jax.ShapeDtypeStruct((C, T, H, W), frames.dtype),
    )(frames)


def kernel(frames):
    C, T, H, W = frames.shape
    n = T // _ALPHA
    table = frames.reshape(C * T * H, W)
    slow2d = _make_sc_gather(C, T, H, W, frames.dtype)(table)
    return (slow2d.reshape(C, n, H, W), _tc_copy(frames))
